# baseline (speedup 1.0000x reference)
import jax
import jax.numpy as jnp
from jax import lax
from jax.experimental import pallas as pl
from jax.experimental.pallas import tpu as pltpu
from jax.experimental.pallas import tpu_sc as plsc

_NUM_HISTOGRAM = 1000
_EMBED = 16
_LOWER = 0.0
_STEP = (1.0 - 0.0) / _NUM_HISTOGRAM

_B, _L = 16384, 200
_NC, _NS = 2, 16
_NW = _NC * _NS
_BBLK = _B // _NW
_LCH = 4
_NCHUNK = _L // _LCH
_LANES = 16
_NG = _BBLK // _LANES


def _body(scores_hbm, table_hbm, out_hbm, s_bufs, rows_bufs, table_v, sem_s, sem_o):
    pltpu.sync_copy(table_hbm, table_v)
    wid = lax.axis_index("s") * _NC + lax.axis_index("c")
    b0 = wid * _BBLK

    def start_s(ci, buf):
        pltpu.async_copy(
            scores_hbm.at[pl.ds(ci * _LCH, _LCH), pl.ds(b0, _BBLK)], buf, sem_s
        )

    def wait_s(buf):
        pltpu.make_async_copy(
            scores_hbm.at[pl.ds(0, _LCH), pl.ds(b0, _BBLK)], buf, sem_s
        ).wait()

    def start_o(ci, buf):
        pltpu.async_copy(
            buf,
            out_hbm.at[pl.ds(ci * _LCH, _LCH), pl.ds(0, _EMBED), pl.ds(b0, _BBLK)],
            sem_o,
        )

    def wait_o(buf):
        pltpu.make_async_copy(
            buf,
            out_hbm.at[pl.ds(0, _LCH), pl.ds(0, _EMBED), pl.ds(b0, _BBLK)],
            sem_o,
        ).wait()

    def compute(s_v, rows_v):
        @plsc.parallel_loop(0, _LCH * _NG, 1, unroll=4)
        def _loop(i):
            li = lax.shift_right_logical(i, 5)
            g = lax.bitwise_and(i, _NG - 1)
            s = s_v[li, pl.ds(g * _LANES, _LANES)]
            base = ((s - _LOWER) / _STEP).astype(jnp.int32) * _EMBED
            vals = [
                plsc.load_gather(table_v, [base + col])
                for col in range(_EMBED)
            ]
            for col in range(_EMBED):
                rows_v[li, col, pl.ds(g * _LANES, _LANES)] = vals[col]

    start_s(0, s_bufs[0])
    start_s(1, s_bufs[1])
    for k in range(2):
        wait_s(s_bufs[k])
        compute(s_bufs[k], rows_bufs[k])
        start_o(k, rows_bufs[k])
        start_s(k + 2, s_bufs[k])

    def pair_body(pi, carry):
        for k in range(2):
            ci = pi * 2 + k
            wait_s(s_bufs[k])
            wait_o(rows_bufs[k])
            compute(s_bufs[k], rows_bufs[k])
            start_o(ci, rows_bufs[k])

            @pl.when(ci + 2 < _NCHUNK)
            def _():
                start_s(ci + 2, s_bufs[k])

        return carry

    lax.fori_loop(1, _NCHUNK // 2, pair_body, 0)
    wait_o(rows_bufs[0])
    wait_o(rows_bufs[1])


def kernel(scores, table):
    f = pl.kernel(
        _body,
        out_type=jax.ShapeDtypeStruct((_L, _EMBED, _B), jnp.float32),
        mesh=plsc.VectorSubcoreMesh(core_axis_name="c", subcore_axis_name="s"),
        compiler_params=pltpu.CompilerParams(needs_layout_passes=False),
        scratch_types=[
            [pltpu.VMEM((_LCH, _BBLK), jnp.float32) for _ in range(2)],
            [pltpu.VMEM((_LCH, _EMBED, _BBLK), jnp.float32) for _ in range(2)],
            pltpu.VMEM((_NUM_HISTOGRAM * _EMBED,), jnp.float32),
            pltpu.SemaphoreType.DMA,
            pltpu.SemaphoreType.DMA,
        ],
    )
    out_t = f(scores.T, table.reshape(_NUM_HISTOGRAM * _EMBED))
    return jnp.transpose(out_t, (2, 0, 1))


# speedup vs baseline: 3.9863x; 3.9863x over previous
import jax
import jax.numpy as jnp
from jax import lax
from jax.experimental import pallas as pl
from jax.experimental.pallas import tpu as pltpu
from jax.experimental.pallas import tpu_sc as plsc

_NUM_HISTOGRAM = 1000
_EMBED = 16
_LOWER = 0.0
_STEP = (1.0 - 0.0) / _NUM_HISTOGRAM

_B, _L = 16384, 200
_NC, _NS = 2, 16
_NW = _NC * _NS
_BBLK = _B // _NW
_LCH = 4
_NCHUNK = _L // _LCH
_LANES = 16
_NG = _BBLK // _LANES


_TSTRIDE = 17


def _body(scores_hbm, table_hbm, out_hbm, s_bufs, rows_bufs, table_v, stage_v,
          sem_s, sem_o):
    pltpu.sync_copy(table_hbm, stage_v)

    @plsc.parallel_loop(0, _NUM_HISTOGRAM, 1, unroll=4)
    def _init(r):
        table_v[pl.ds(r * _TSTRIDE, _EMBED)] = stage_v[pl.ds(r * _EMBED, _EMBED)]

    wid = lax.axis_index("s") * _NC + lax.axis_index("c")
    b0 = wid * _BBLK

    def start_s(ci, buf):
        pltpu.async_copy(
            scores_hbm.at[pl.ds(ci * _LCH, _LCH), pl.ds(b0, _BBLK)], buf, sem_s
        )

    def wait_s(buf):
        pltpu.make_async_copy(
            scores_hbm.at[pl.ds(0, _LCH), pl.ds(b0, _BBLK)], buf, sem_s
        ).wait()

    def start_o(ci, buf):
        pltpu.async_copy(
            buf,
            out_hbm.at[pl.ds(ci * _LCH, _LCH), pl.ds(0, _EMBED), pl.ds(b0, _BBLK)],
            sem_o,
        )

    def wait_o(buf):
        pltpu.make_async_copy(
            buf,
            out_hbm.at[pl.ds(0, _LCH), pl.ds(0, _EMBED), pl.ds(b0, _BBLK)],
            sem_o,
        ).wait()

    def compute(s_v, rows_v):
        @plsc.parallel_loop(0, _LCH * _NG, 1, unroll=2)
        def _loop(i):
            li = lax.shift_right_logical(i, 5)
            g = lax.bitwise_and(i, _NG - 1)
            s = s_v[li, pl.ds(g * _LANES, _LANES)]
            base = ((s - _LOWER) / _STEP).astype(jnp.int32) * _TSTRIDE
            vals = [
                plsc.load_gather(table_v, [base + col])
                for col in range(_EMBED)
            ]
            for col in range(_EMBED):
                rows_v[li, col, pl.ds(g * _LANES, _LANES)] = vals[col]

    start_s(0, s_bufs[0])
    start_s(1, s_bufs[1])
    for k in range(2):
        wait_s(s_bufs[k])
        compute(s_bufs[k], rows_bufs[k])
        start_o(k, rows_bufs[k])
        start_s(k + 2, s_bufs[k])

    def pair_body(pi, carry):
        for k in range(2):
            ci = pi * 2 + k
            wait_s(s_bufs[k])
            wait_o(rows_bufs[k])
            compute(s_bufs[k], rows_bufs[k])
            start_o(ci, rows_bufs[k])

            @pl.when(ci + 2 < _NCHUNK)
            def _():
                start_s(ci + 2, s_bufs[k])

        return carry

    lax.fori_loop(1, _NCHUNK // 2, pair_body, 0)
    wait_o(rows_bufs[0])
    wait_o(rows_bufs[1])


def kernel(scores, table):
    f = pl.kernel(
        _body,
        out_type=jax.ShapeDtypeStruct((_L, _EMBED, _B), jnp.float32),
        mesh=plsc.VectorSubcoreMesh(core_axis_name="c", subcore_axis_name="s"),
        compiler_params=pltpu.CompilerParams(needs_layout_passes=False),
        scratch_types=[
            [pltpu.VMEM((_LCH, _BBLK), jnp.float32) for _ in range(2)],
            [pltpu.VMEM((_LCH, _EMBED, _BBLK), jnp.float32) for _ in range(2)],
            pltpu.VMEM((_NUM_HISTOGRAM * _TSTRIDE,), jnp.float32),
            pltpu.VMEM((_NUM_HISTOGRAM * _EMBED,), jnp.float32),
            pltpu.SemaphoreType.DMA,
            pltpu.SemaphoreType.DMA,
        ],
    )
    out_t = f(scores.T, table.reshape(_NUM_HISTOGRAM * _EMBED))
    return jnp.transpose(out_t, (2, 0, 1))
